# baseline (speedup 1.0000x reference)
import functools

import jax
import jax.numpy as jnp
from jax import lax
from jax.experimental import pallas as pl
from jax.experimental.pallas import tpu as pltpu
from jax.experimental.pallas import tpu_sc as plsc

NTOKEN = 1000000
EMB_DIM = 32
BATCH = 16384
HIST = 50

B = BATCH * HIST
NC, NS = 2, 16
NW = NC * NS
BPB = BATCH // NW
CB = 64
NCHUNK = BPB // CB
HG = 2
NHG = HIST // HG
LP = CB + 1

_mesh = plsc.VectorSubcoreMesh(core_axis_name="c", subcore_axis_name="s")


@functools.partial(
    pl.kernel,
    mesh=_mesh,
    out_type=jax.ShapeDtypeStruct(
        (HIST, EMB_DIM // 8, BATCH // 128, 8, 128), jnp.float32),
    scratch_types=[
        pltpu.VMEM((2, HIST, CB), jnp.int32),
        pltpu.VMEM((HIST, CB, EMB_DIM), jnp.float32),
        pltpu.VMEM((2, HG, EMB_DIM // 8, 8, LP), jnp.float32),
        pltpu.SemaphoreType.DMA,
        pltpu.SemaphoreType.DMA,
        pltpu.SemaphoreType.DMA,
        pltpu.SemaphoreType.DMA,
    ],
    compiler_params=pltpu.CompilerParams(
        use_tc_tiling_on_sc=False, needs_layout_passes=False),
)
def _gather_kernel(xt_hbm, table_hbm, out_hbm, idx_v, gbuf, tbuf,
                   isem, gsem, ssem0, ssem1):
    wid = lax.axis_index("s") * NC + lax.axis_index("c")
    b0 = wid * BPB
    iota16 = lax.iota(jnp.int32, 16)
    s_vec = lax.rem(iota16, 8)
    tr0 = lax.div(iota16, 8)
    tr1 = tr0 + 2

    def store_src(tb):
        return tbuf.at[tb, :, :, :, pl.ds(0, CB)]

    def store_dst(g, tc, l0):
        return out_hbm.at[pl.ds(g * HG, HG), :, tc, :, pl.ds(l0, CB)]

    ssems = (ssem0, ssem1)

    def fire_idx(c, ib):
        pltpu.async_copy(
            xt_hbm.at[:, pl.ds(b0 + c * CB, CB)], idx_v.at[ib], isem)

    fire_idx(0, 0)

    def chunk(c, carry):
        ib = c % 2
        pltpu.make_async_copy(
            xt_hbm.at[:, pl.ds(b0, CB)], idx_v.at[ib], isem).wait()

        @pl.loop(0, HIST)
        def _(j):
            pltpu.async_copy(
                table_hbm.at[idx_v.at[ib, j]], gbuf.at[j], gsem)

        @pl.when(c + 1 < NCHUNK)
        def _():
            fire_idx(c + 1, 1 - ib)

        @pl.loop(0, HIST)
        def _(j):
            pltpu.make_async_copy(
                table_hbm.at[idx_v.at[ib, j]], gbuf.at[j], gsem).wait()

        tc = wid * (BPB // 128) + c // 2
        l0 = (c % 2) * CB
        for g in range(NHG):
            tb = g % 2
            if g >= 2:
                pltpu.make_async_copy(
                    store_src(tb), store_dst(g - 2, tc, l0), ssems[tb]).wait()
            else:
                @pl.when(c > 0)
                def _():
                    pltpu.make_async_copy(
                        store_src(tb), store_dst(0, 0, 0), ssems[tb]).wait()

            g5 = g * HG
            tbv = jnp.full((16,), tb, jnp.int32)

            @plsc.parallel_loop(0, HG * CB, unroll=16)
            def _(i):
                hl = lax.div(i, CB)
                bl = lax.rem(i, CB)
                h = g5 + hl
                v0 = gbuf[h, bl, pl.ds(0, 16)]
                v1 = gbuf[h, bl, pl.ds(16, 16)]
                hlv = jnp.full((16,), hl, jnp.int32)
                blv = jnp.full((16,), bl, jnp.int32)
                plsc.store_scatter(tbuf, [tbv, hlv, tr0, s_vec, blv], v0)
                plsc.store_scatter(tbuf, [tbv, hlv, tr1, s_vec, blv], v1)

            pltpu.async_copy(store_src(tb), store_dst(g, tc, l0), ssems[tb])
        return carry

    lax.fori_loop(0, NCHUNK, chunk, 0)
    pltpu.make_async_copy(store_src(0), store_dst(0, 0, 0), ssem0).wait()
    pltpu.make_async_copy(store_src(1), store_dst(0, 0, 0), ssem1).wait()


def kernel(x, table):
    xt = x.T.astype(jnp.int32)
    out5 = _gather_kernel(xt, table)
    return out5.transpose(2, 4, 0, 1, 3).reshape(BATCH, HIST, EMB_DIM)


# speedup vs baseline: 1.0397x; 1.0397x over previous
import functools

import jax
import jax.numpy as jnp
from jax import lax
from jax.experimental import pallas as pl
from jax.experimental.pallas import tpu as pltpu
from jax.experimental.pallas import tpu_sc as plsc

NTOKEN = 1000000
EMB_DIM = 32
BATCH = 16384
HIST = 50

B = BATCH * HIST
NC, NS = 2, 16
NW = NC * NS
BPB = BATCH // NW
CB = 64
NCHUNK = BPB // CB
HG = 2
NHG = HIST // HG
LP = CB + 1

_mesh = plsc.VectorSubcoreMesh(core_axis_name="c", subcore_axis_name="s")


@functools.partial(
    pl.kernel,
    mesh=_mesh,
    out_type=jax.ShapeDtypeStruct(
        (HIST, EMB_DIM // 8, BATCH // 128, 8, 128), jnp.float32),
    scratch_types=[
        pltpu.VMEM((2, HIST, CB), jnp.int32),
        pltpu.VMEM((HIST, CB, EMB_DIM), jnp.float32),
        pltpu.VMEM((2, HG, EMB_DIM // 8, 8, LP), jnp.float32),
        pltpu.SemaphoreType.DMA,
        pltpu.SemaphoreType.DMA,
        pltpu.SemaphoreType.DMA,
        pltpu.SemaphoreType.DMA,
    ],
    compiler_params=pltpu.CompilerParams(
        use_tc_tiling_on_sc=False, needs_layout_passes=False),
)
def _gather_kernel(xt_hbm, table_hbm, out_hbm, idx_v, gbuf, tbuf,
                   isem, gsem, ssem0, ssem1):
    wid = lax.axis_index("s") * NC + lax.axis_index("c")
    b0 = wid * BPB
    iota16 = lax.iota(jnp.int32, 16)
    s_vec = lax.rem(iota16, 8)
    tr0 = lax.div(iota16, 8)
    tr1 = tr0 + 2

    def store_src(tb):
        return tbuf.at[tb, :, :, :, pl.ds(0, CB)]

    def store_dst(g, tc, l0):
        return out_hbm.at[pl.ds(g * HG, HG), :, tc, :, pl.ds(l0, CB)]

    ssems = (ssem0, ssem1)

    def fire_idx(c, ib):
        pltpu.async_copy(
            xt_hbm.at[:, pl.ds(b0 + c * CB, CB)], idx_v.at[ib], isem)

    fire_idx(0, 0)

    def chunk(c, carry):
        ib = c % 2
        pltpu.make_async_copy(
            xt_hbm.at[:, pl.ds(b0, CB)], idx_v.at[ib], isem).wait()

        @pl.loop(0, HIST)
        def _(j):
            pltpu.async_copy(
                table_hbm.at[idx_v.at[ib, j]], gbuf.at[j], gsem)

        @pl.when(c + 1 < NCHUNK)
        def _():
            fire_idx(c + 1, 1 - ib)

        @pl.loop(0, HIST)
        def _(j):
            pltpu.make_async_copy(
                table_hbm.at[idx_v.at[ib, j]], gbuf.at[j], gsem).wait()

        tc = wid * (BPB // 128) + c // 2
        l0 = (c % 2) * CB
        for g in range(NHG):
            tb = g % 2
            if g >= 2:
                pltpu.make_async_copy(
                    store_src(tb), store_dst(g - 2, tc, l0), ssems[tb]).wait()
            else:
                @pl.when(c > 0)
                def _():
                    pltpu.make_async_copy(
                        store_src(tb), store_dst(0, 0, 0), ssems[tb]).wait()

            g5 = g * HG
            tbv = jnp.full((16,), tb, jnp.int32)

            @plsc.parallel_loop(0, HG * CB, unroll=8)
            def _(i):
                hl = lax.div(i, CB)
                bl = lax.rem(i, CB)
                h = g5 + hl
                v0 = gbuf[h, bl, pl.ds(0, 16)]
                v1 = gbuf[h, bl, pl.ds(16, 16)]
                hlv = jnp.full((16,), hl, jnp.int32)
                blv = jnp.full((16,), bl, jnp.int32)
                plsc.store_scatter(tbuf, [tbv, hlv, tr0, s_vec, blv], v0)
                plsc.store_scatter(tbuf, [tbv, hlv, tr1, s_vec, blv], v1)

            pltpu.async_copy(store_src(tb), store_dst(g, tc, l0), ssems[tb])
        return carry

    lax.fori_loop(0, NCHUNK, chunk, 0)
    pltpu.make_async_copy(store_src(0), store_dst(0, 0, 0), ssem0).wait()
    pltpu.make_async_copy(store_src(1), store_dst(0, 0, 0), ssem1).wait()


def kernel(x, table):
    xt = x.T.astype(jnp.int32)
    out5 = _gather_kernel(xt, table)
    return out5.transpose(2, 4, 0, 1, 3).reshape(BATCH, HIST, EMB_DIM)
